# baseline (speedup 1.0000x reference)
import functools

import jax
import jax.numpy as jnp
from jax import lax
from jax.experimental import pallas as pl
from jax.experimental.pallas import tpu as pltpu
from jax.experimental.pallas import tpu_sc as plsc

NC, NS = 2, 16
NW = NC * NS
B = 16384
D = 64
DP = 2 * D
BPW = B // NW
CH = 128
NCH = BPW // CH
LANES = 16
NG = CH // LANES


def _body(uh_hbm, ih_hbm, jh_hbm, user_hbm, ii_hbm, ij_hbm,
          uw_hbm, iw_hbm, out_i_hbm, out_j_hbm,
          uhalf, ihalf, jhalf, uidx, iidx, jidx,
          urows, irows, jrows, oi, oj, sem):
    wid = lax.axis_index("s") * NC + lax.axis_index("c")

    pltpu.sync_copy(uh_hbm.at[wid], uhalf)
    pltpu.sync_copy(ih_hbm.at[wid], ihalf)
    pltpu.sync_copy(jh_hbm.at[wid], jhalf)
    pltpu.sync_copy(user_hbm.at[wid], uidx)
    pltpu.sync_copy(ii_hbm.at[wid], iidx)
    pltpu.sync_copy(ij_hbm.at[wid], jidx)

    iota = lax.iota(jnp.int32, LANES)

    def chunk(c, carry):
        cps = [
            pltpu.async_copy(uw_hbm.at[uhalf.at[c]], urows, sem),
            pltpu.async_copy(iw_hbm.at[ihalf.at[c]], irows, sem),
            pltpu.async_copy(iw_hbm.at[jhalf.at[c]], jrows, sem),
        ]
        for cp in cps:
            cp.wait()
        out_base = c * CH

        def group(g, carry2):
            s = g * LANES
            rows = s + iota
            u_col = (uidx[c, pl.ds(s, LANES)] & 1) * D
            i_col = (iidx[c, pl.ds(s, LANES)] & 1) * D
            j_col = (jidx[c, pl.ds(s, LANES)] & 1) * D
            u0 = plsc.load_gather(urows, [rows, u_col])
            acc_i = u0 * plsc.load_gather(irows, [rows, i_col])
            acc_j = u0 * plsc.load_gather(jrows, [rows, j_col])
            for d in range(1, D):
                ud = plsc.load_gather(urows, [rows, u_col + d])
                acc_i = acc_i + ud * plsc.load_gather(irows, [rows, i_col + d])
                acc_j = acc_j + ud * plsc.load_gather(jrows, [rows, j_col + d])
            oi[pl.ds(out_base + s, LANES)] = acc_i
            oj[pl.ds(out_base + s, LANES)] = acc_j
            return carry2

        lax.fori_loop(0, NG, group, 0)
        return carry

    lax.fori_loop(0, NCH, chunk, 0)

    pltpu.sync_copy(oi, out_i_hbm.at[pl.ds(wid * BPW, BPW)])
    pltpu.sync_copy(oj, out_j_hbm.at[pl.ds(wid * BPW, BPW)])


@jax.jit
def _sc_bpr(uh3, ih3, jh3, user3, ii3, ij3, uw2, iw2):
    f32 = jnp.float32
    call = pl.kernel(
        _body,
        out_type=(jax.ShapeDtypeStruct((B,), f32),
                  jax.ShapeDtypeStruct((B,), f32)),
        mesh=plsc.VectorSubcoreMesh(
            core_axis_name="c", subcore_axis_name="s",
            num_cores=NC, num_subcores=NS),
        scratch_types=[
            pltpu.VMEM((NCH, CH), jnp.int32),
            pltpu.VMEM((NCH, CH), jnp.int32),
            pltpu.VMEM((NCH, CH), jnp.int32),
            pltpu.VMEM((NCH, CH), jnp.int32),
            pltpu.VMEM((NCH, CH), jnp.int32),
            pltpu.VMEM((NCH, CH), jnp.int32),
            pltpu.VMEM((CH, DP), f32),
            pltpu.VMEM((CH, DP), f32),
            pltpu.VMEM((CH, DP), f32),
            pltpu.VMEM((BPW,), f32),
            pltpu.VMEM((BPW,), f32),
            pltpu.SemaphoreType.DMA,
        ],
        compiler_params=pltpu.CompilerParams(needs_layout_passes=False),
    )
    return call(uh3, ih3, jh3, user3, ii3, ij3, uw2, iw2)


TBLK = 8192
TGRID = (1000000 + TBLK - 1) // TBLK


def _transpose_body(eye_ref, xt_ref, out_ref):
    out_ref[...] = jax.lax.dot_general(
        xt_ref[...], eye_ref[...], (((0,), (0,)), ((), ())),
        preferred_element_type=jnp.float32)


def _tc_transpose(xt):
    eye = jnp.eye(D, dtype=jnp.float32)
    return pl.pallas_call(
        _transpose_body,
        grid=(TGRID,),
        in_specs=[
            pl.BlockSpec((D, D), lambda i: (0, 0)),
            pl.BlockSpec((D, TBLK), lambda i: (0, i)),
        ],
        out_specs=pl.BlockSpec((TBLK, D), lambda i: (i, 0)),
        out_shape=jax.ShapeDtypeStruct((1000000, D), jnp.float32),
        compiler_params=pltpu.CompilerParams(
            fuse_transposed_lhs_in_matmul=True),
    )(eye, xt)


def kernel(user, item_i, item_j, embed_user_w, embed_item_w):
    user = user.astype(jnp.int32)
    item_i = item_i.astype(jnp.int32)
    item_j = item_j.astype(jnp.int32)
    uh3 = (user >> 1).reshape(NW, NCH, CH)
    ih3 = (item_i >> 1).reshape(NW, NCH, CH)
    jh3 = (item_j >> 1).reshape(NW, NCH, CH)
    user3 = user.reshape(NW, NCH, CH)
    ii3 = item_i.reshape(NW, NCH, CH)
    ij3 = item_j.reshape(NW, NCH, CH)
    uw2 = _tc_transpose(embed_user_w.T).reshape(-1, DP)
    iw2 = _tc_transpose(embed_item_w.T).reshape(-1, DP)
    return _sc_bpr(uh3, ih3, jh3, user3, ii3, ij3, uw2, iw2)


# speedup vs baseline: 1.0460x; 1.0460x over previous
import functools

import jax
import jax.numpy as jnp
from jax import lax
from jax.experimental import pallas as pl
from jax.experimental.pallas import tpu as pltpu
from jax.experimental.pallas import tpu_sc as plsc

NC, NS = 2, 16
NW = NC * NS
B = 16384
D = 64
DP = 2 * D
BPW = B // NW
CH = 128
NCH = BPW // CH
LANES = 16
NG = CH // LANES


def _body(uh_hbm, ih_hbm, jh_hbm, user_hbm, ii_hbm, ij_hbm,
          uw_hbm, iw_hbm, out_i_hbm, out_j_hbm,
          uhalf, ihalf, jhalf, uidx, iidx, jidx,
          urows, irows, jrows, oi, oj, sem):
    wid = lax.axis_index("s") * NC + lax.axis_index("c")

    pltpu.sync_copy(uh_hbm.at[wid], uhalf)
    pltpu.sync_copy(ih_hbm.at[wid], ihalf)
    pltpu.sync_copy(jh_hbm.at[wid], jhalf)
    pltpu.sync_copy(user_hbm.at[wid], uidx)
    pltpu.sync_copy(ii_hbm.at[wid], iidx)
    pltpu.sync_copy(ij_hbm.at[wid], jidx)

    iota = lax.iota(jnp.int32, LANES)

    def chunk(c, carry):
        cps = [
            pltpu.async_copy(uw_hbm.at[uhalf.at[c]], urows, sem),
            pltpu.async_copy(iw_hbm.at[ihalf.at[c]], irows, sem),
            pltpu.async_copy(iw_hbm.at[jhalf.at[c]], jrows, sem),
        ]
        for cp in cps:
            cp.wait()
        out_base = c * CH

        def group(g, carry2):
            s = g * LANES
            rows = s + iota
            u_col = (uidx[c, pl.ds(s, LANES)] & 1) * D
            i_col = (iidx[c, pl.ds(s, LANES)] & 1) * D
            j_col = (jidx[c, pl.ds(s, LANES)] & 1) * D
            u0 = plsc.load_gather(urows, [rows, u_col])
            acc_i = u0 * plsc.load_gather(irows, [rows, i_col])
            acc_j = u0 * plsc.load_gather(jrows, [rows, j_col])
            for d in range(1, D):
                ud = plsc.load_gather(urows, [rows, u_col + d])
                acc_i = acc_i + ud * plsc.load_gather(irows, [rows, i_col + d])
                acc_j = acc_j + ud * plsc.load_gather(jrows, [rows, j_col + d])
            oi[pl.ds(out_base + s, LANES)] = acc_i
            oj[pl.ds(out_base + s, LANES)] = acc_j
            return carry2

        lax.fori_loop(0, NG, group, 0)
        return carry

    lax.fori_loop(0, NCH, chunk, 0)

    pltpu.sync_copy(oi, out_i_hbm.at[pl.ds(wid * BPW, BPW)])
    pltpu.sync_copy(oj, out_j_hbm.at[pl.ds(wid * BPW, BPW)])


@jax.jit
def _sc_bpr(uh3, ih3, jh3, user3, ii3, ij3, uw2, iw2):
    f32 = jnp.float32
    call = pl.kernel(
        _body,
        out_type=(jax.ShapeDtypeStruct((B,), f32),
                  jax.ShapeDtypeStruct((B,), f32)),
        mesh=plsc.VectorSubcoreMesh(
            core_axis_name="c", subcore_axis_name="s",
            num_cores=NC, num_subcores=NS),
        scratch_types=[
            pltpu.VMEM((NCH, CH), jnp.int32),
            pltpu.VMEM((NCH, CH), jnp.int32),
            pltpu.VMEM((NCH, CH), jnp.int32),
            pltpu.VMEM((NCH, CH), jnp.int32),
            pltpu.VMEM((NCH, CH), jnp.int32),
            pltpu.VMEM((NCH, CH), jnp.int32),
            pltpu.VMEM((CH, DP), f32),
            pltpu.VMEM((CH, DP), f32),
            pltpu.VMEM((CH, DP), f32),
            pltpu.VMEM((BPW,), f32),
            pltpu.VMEM((BPW,), f32),
            pltpu.SemaphoreType.DMA,
        ],
        compiler_params=pltpu.CompilerParams(needs_layout_passes=False),
    )
    return call(uh3, ih3, jh3, user3, ii3, ij3, uw2, iw2)


TBLK = 32768
TGRID = (1000000 + TBLK - 1) // TBLK


def _transpose_body(eye_ref, xt_ref, out_ref):
    out_ref[...] = jax.lax.dot_general(
        xt_ref[...], eye_ref[...], (((0,), (0,)), ((), ())),
        preferred_element_type=jnp.float32)


def _tc_transpose(xt):
    eye = jnp.eye(D, dtype=jnp.float32)
    return pl.pallas_call(
        _transpose_body,
        grid=(TGRID,),
        in_specs=[
            pl.BlockSpec((D, D), lambda i: (0, 0)),
            pl.BlockSpec((D, TBLK), lambda i: (0, i)),
        ],
        out_specs=pl.BlockSpec((TBLK, D), lambda i: (i, 0)),
        out_shape=jax.ShapeDtypeStruct((1000000, D), jnp.float32),
        compiler_params=pltpu.CompilerParams(
            fuse_transposed_lhs_in_matmul=True),
    )(eye, xt)


def kernel(user, item_i, item_j, embed_user_w, embed_item_w):
    user = user.astype(jnp.int32)
    item_i = item_i.astype(jnp.int32)
    item_j = item_j.astype(jnp.int32)
    uh3 = (user >> 1).reshape(NW, NCH, CH)
    ih3 = (item_i >> 1).reshape(NW, NCH, CH)
    jh3 = (item_j >> 1).reshape(NW, NCH, CH)
    user3 = user.reshape(NW, NCH, CH)
    ii3 = item_i.reshape(NW, NCH, CH)
    ij3 = item_j.reshape(NW, NCH, CH)
    uw2 = _tc_transpose(embed_user_w.T).reshape(-1, DP)
    iw2 = _tc_transpose(embed_item_w.T).reshape(-1, DP)
    return _sc_bpr(uh3, ih3, jh3, user3, ii3, ij3, uw2, iw2)


# speedup vs baseline: 1.0966x; 1.0484x over previous
import functools

import jax
import jax.numpy as jnp
from jax import lax
from jax.experimental import pallas as pl
from jax.experimental.pallas import tpu as pltpu
from jax.experimental.pallas import tpu_sc as plsc

NC, NS = 2, 16
NW = NC * NS
B = 16384
D = 64
DP = 2 * D
BPW = B // NW
CH = 128
NCH = BPW // CH
LANES = 16
NG = CH // LANES


def _body(uh_hbm, ih_hbm, jh_hbm, user_hbm, ii_hbm, ij_hbm,
          uw_hbm, iw_hbm, out_i_hbm, out_j_hbm,
          uhalf, ihalf, jhalf, uidx, iidx, jidx,
          urows, irows, jrows, oi, oj, sem):
    wid = lax.axis_index("s") * NC + lax.axis_index("c")

    pltpu.sync_copy(uh_hbm.at[wid], uhalf)
    pltpu.sync_copy(ih_hbm.at[wid], ihalf)
    pltpu.sync_copy(jh_hbm.at[wid], jhalf)
    pltpu.sync_copy(user_hbm.at[wid], uidx)
    pltpu.sync_copy(ii_hbm.at[wid], iidx)
    pltpu.sync_copy(ij_hbm.at[wid], jidx)

    iota = lax.iota(jnp.int32, LANES)

    def chunk(c, carry):
        cps = [
            pltpu.async_copy(uw_hbm.at[uhalf.at[c]], urows, sem),
            pltpu.async_copy(iw_hbm.at[ihalf.at[c]], irows, sem),
            pltpu.async_copy(iw_hbm.at[jhalf.at[c]], jrows, sem),
        ]
        for cp in cps:
            cp.wait()
        out_base = c * CH

        def group(g, carry2):
            s = g * LANES
            rows = s + iota
            u_col = (uidx[c, pl.ds(s, LANES)] & 1) * D
            i_col = (iidx[c, pl.ds(s, LANES)] & 1) * D
            j_col = (jidx[c, pl.ds(s, LANES)] & 1) * D
            u0 = plsc.load_gather(urows, [rows, u_col])
            acc_i = u0 * plsc.load_gather(irows, [rows, i_col])
            acc_j = u0 * plsc.load_gather(jrows, [rows, j_col])
            for d in range(1, D):
                ud = plsc.load_gather(urows, [rows, u_col + d])
                acc_i = acc_i + ud * plsc.load_gather(irows, [rows, i_col + d])
                acc_j = acc_j + ud * plsc.load_gather(jrows, [rows, j_col + d])
            oi[pl.ds(out_base + s, LANES)] = acc_i
            oj[pl.ds(out_base + s, LANES)] = acc_j
            return carry2

        lax.fori_loop(0, NG, group, 0)
        return carry

    lax.fori_loop(0, NCH, chunk, 0)

    pltpu.sync_copy(oi, out_i_hbm.at[pl.ds(wid * BPW, BPW)])
    pltpu.sync_copy(oj, out_j_hbm.at[pl.ds(wid * BPW, BPW)])


@jax.jit
def _sc_bpr(uh3, ih3, jh3, user3, ii3, ij3, uw2, iw2):
    f32 = jnp.float32
    call = pl.kernel(
        _body,
        out_type=(jax.ShapeDtypeStruct((B,), f32),
                  jax.ShapeDtypeStruct((B,), f32)),
        mesh=plsc.VectorSubcoreMesh(
            core_axis_name="c", subcore_axis_name="s",
            num_cores=NC, num_subcores=NS),
        scratch_types=[
            pltpu.VMEM((NCH, CH), jnp.int32),
            pltpu.VMEM((NCH, CH), jnp.int32),
            pltpu.VMEM((NCH, CH), jnp.int32),
            pltpu.VMEM((NCH, CH), jnp.int32),
            pltpu.VMEM((NCH, CH), jnp.int32),
            pltpu.VMEM((NCH, CH), jnp.int32),
            pltpu.VMEM((CH, DP), f32),
            pltpu.VMEM((CH, DP), f32),
            pltpu.VMEM((CH, DP), f32),
            pltpu.VMEM((BPW,), f32),
            pltpu.VMEM((BPW,), f32),
            pltpu.SemaphoreType.DMA,
        ],
        compiler_params=pltpu.CompilerParams(needs_layout_passes=False),
    )
    return call(uh3, ih3, jh3, user3, ii3, ij3, uw2, iw2)


TBLK = 32768
TGRID = (1000000 + TBLK - 1) // TBLK


def _transpose_body(eye_ref, xt_ref, out_ref):
    out_ref[...] = jax.lax.dot_general(
        xt_ref[...], eye_ref[...], (((0,), (0,)), ((), ())),
        preferred_element_type=jnp.float32)


def _tc_transpose(xt):
    eye = jnp.eye(D, dtype=jnp.float32)
    return pl.pallas_call(
        _transpose_body,
        grid=(TGRID,),
        in_specs=[
            pl.BlockSpec((D, D), lambda i: (0, 0)),
            pl.BlockSpec((D, TBLK), lambda i: (0, i)),
        ],
        out_specs=pl.BlockSpec((TBLK, D), lambda i: (i, 0)),
        out_shape=jax.ShapeDtypeStruct((1000000, D), jnp.float32),
        compiler_params=pltpu.CompilerParams(
            fuse_transposed_lhs_in_matmul=True),
    )(eye, xt)


def kernel(user, item_i, item_j, embed_user_w, embed_item_w):
    user = user.astype(jnp.int32)
    item_i = item_i.astype(jnp.int32)
    item_j = item_j.astype(jnp.int32)
    uh3 = (user >> 1).reshape(NW, NCH, CH)
    ih3 = (item_i >> 1).reshape(NW, NCH, CH)
    jh3 = (item_j >> 1).reshape(NW, NCH, CH)
    user3 = user.reshape(NW, NCH, CH)
    ii3 = item_i.reshape(NW, NCH, CH)
    ij3 = item_j.reshape(NW, NCH, CH)
    uw2 = _tc_transpose(embed_user_w.T).reshape(-1, DP)
    iw2 = embed_item_w.reshape(-1, DP)
    return _sc_bpr(uh3, ih3, jh3, user3, ii3, ij3, uw2, iw2)


# speedup vs baseline: 1.2464x; 1.1365x over previous
import functools

import jax
import jax.numpy as jnp
from jax import lax
from jax.experimental import pallas as pl
from jax.experimental.pallas import tpu as pltpu
from jax.experimental.pallas import tpu_sc as plsc

NC, NS = 2, 16
NW = NC * NS
B = 16384
D = 64
BPW = B // NW
CH = 128
NCH = BPW // CH
LANES = 16


def _body(user_hbm, ii_hbm, ij_hbm, uw_hbm, iw_hbm, out_i_hbm, out_j_hbm,
          uidx, iidx, jidx, urows, irows, jrows, ti, tj, oi, oj, sem):
    wid = lax.axis_index("s") * NC + lax.axis_index("c")

    pltpu.sync_copy(user_hbm.at[wid], uidx)
    pltpu.sync_copy(ii_hbm.at[wid], iidx)
    pltpu.sync_copy(ij_hbm.at[wid], jidx)

    copies = []
    for c in range(NCH):
        copies.append(pltpu.async_copy(
            uw_hbm.at[uidx.at[c]], urows.at[pl.ds(c * CH, CH)], sem))
        copies.append(pltpu.async_copy(
            iw_hbm.at[iidx.at[c]], irows.at[pl.ds(c * CH, CH)], sem))
        copies.append(pltpu.async_copy(
            iw_hbm.at[jidx.at[c]], jrows.at[pl.ds(c * CH, CH)], sem))
    for cp in copies:
        cp.wait()

    iota = lax.iota(jnp.int32, LANES)

    def group(g, carry):
        base_r = g * LANES
        for m in range(LANES):
            r = base_r + m
            acc_i = urows[r, pl.ds(0, LANES)] * irows[r, pl.ds(0, LANES)]
            acc_j = urows[r, pl.ds(0, LANES)] * jrows[r, pl.ds(0, LANES)]
            for k in range(1, D // LANES):
                u = urows[r, pl.ds(k * LANES, LANES)]
                acc_i = acc_i + u * irows[r, pl.ds(k * LANES, LANES)]
                acc_j = acc_j + u * jrows[r, pl.ds(k * LANES, LANES)]
            tidx = iota * LANES + m
            plsc.store_scatter(ti, [tidx], acc_i)
            plsc.store_scatter(tj, [tidx], acc_j)
        si = ti[pl.ds(0, LANES)]
        sj = tj[pl.ds(0, LANES)]
        for m in range(1, LANES):
            si = si + ti[pl.ds(m * LANES, LANES)]
            sj = sj + tj[pl.ds(m * LANES, LANES)]
        oi[pl.ds(base_r, LANES)] = si
        oj[pl.ds(base_r, LANES)] = sj
        return carry

    lax.fori_loop(0, BPW // LANES, group, 0)

    pltpu.sync_copy(oi, out_i_hbm.at[pl.ds(wid * BPW, BPW)])
    pltpu.sync_copy(oj, out_j_hbm.at[pl.ds(wid * BPW, BPW)])


@jax.jit
def _sc_bpr(user3, ii3, ij3, uw, iw):
    f32 = jnp.float32
    call = pl.kernel(
        _body,
        out_type=(jax.ShapeDtypeStruct((B,), f32),
                  jax.ShapeDtypeStruct((B,), f32)),
        mesh=plsc.VectorSubcoreMesh(
            core_axis_name="c", subcore_axis_name="s",
            num_cores=NC, num_subcores=NS),
        scratch_types=[
            pltpu.VMEM((NCH, CH), jnp.int32),
            pltpu.VMEM((NCH, CH), jnp.int32),
            pltpu.VMEM((NCH, CH), jnp.int32),
            pltpu.VMEM((BPW, D), f32),
            pltpu.VMEM((BPW, D), f32),
            pltpu.VMEM((BPW, D), f32),
            pltpu.VMEM((LANES * LANES,), f32),
            pltpu.VMEM((LANES * LANES,), f32),
            pltpu.VMEM((BPW,), f32),
            pltpu.VMEM((BPW,), f32),
            pltpu.SemaphoreType.DMA,
        ],
        compiler_params=pltpu.CompilerParams(
            needs_layout_passes=False, use_tc_tiling_on_sc=False),
    )
    return call(user3, ii3, ij3, uw, iw)


def kernel(user, item_i, item_j, embed_user_w, embed_item_w):
    user3 = user.astype(jnp.int32).reshape(NW, NCH, CH)
    ii3 = item_i.astype(jnp.int32).reshape(NW, NCH, CH)
    ij3 = item_j.astype(jnp.int32).reshape(NW, NCH, CH)
    return _sc_bpr(user3, ii3, ij3, embed_user_w, embed_item_w)
